# baseline (speedup 1.0000x reference)
import functools

import jax
import jax.numpy as jnp
from jax import lax
from jax.experimental import pallas as pl
from jax.experimental.pallas import tpu as pltpu
from jax.experimental.pallas import tpu_sc as plsc

DIM = 32
ROWS = 16384
COLS = 26
B = ROWS * COLS
NW = 32
BPW = B // NW
CH = 832
NCH = BPW // CH
NBUF = 4
NGIF = 3

_mesh = plsc.VectorSubcoreMesh(core_axis_name="c", subcore_axis_name="s")


@functools.partial(
    pl.kernel,
    mesh=_mesh,
    out_type=jax.ShapeDtypeStruct((B, DIM), jnp.float32),
    scratch_types=[
        pltpu.VMEM((BPW,), jnp.int32),
        pltpu.VMEM((NBUF, CH, DIM), jnp.float32),
        pltpu.SemaphoreType.DMA((NBUF,)),
        pltpu.SemaphoreType.DMA((NBUF,)),
    ],
    compiler_params=pltpu.CompilerParams(use_tc_tiling_on_sc=False),
)
def _emb_lookup(x_hbm, table_hbm, out_hbm, idx_v, rows_v, gsem, osem):
    wid = lax.axis_index("s") * 2 + lax.axis_index("c")
    base = wid * BPW

    pltpu.sync_copy(x_hbm.at[pl.ds(base, BPW)], idx_v)

    def fire_gather(j):
        return pltpu.async_copy(
            table_hbm.at[idx_v.at[pl.ds(j * CH, CH)]],
            rows_v.at[j % NBUF],
            gsem.at[j % NBUF],
        )

    gh = [None] * NCH
    oh = [None] * NCH
    for j in range(NGIF):
        gh[j] = fire_gather(j)
    for i in range(NCH):
        b = i % NBUF
        gh[i].wait()
        oh[i] = pltpu.async_copy(
            rows_v.at[b], out_hbm.at[pl.ds(base + i * CH, CH)], osem.at[b]
        )
        j = i + NGIF
        if j < NCH:
            if j >= NBUF:
                oh[j - NBUF].wait()
            gh[j] = fire_gather(j)
    for i in range(max(0, NCH - NBUF), NCH):
        oh[i].wait()


def kernel(x, table):
    t_lin = lax.optimization_barrier(table.reshape(250000, 4 * DIM))
    out = _emb_lookup(x.reshape(B), t_lin.reshape(1000000, DIM))
    out_w = lax.optimization_barrier(out.reshape(B // 4, 4 * DIM))
    return out_w.reshape(ROWS, COLS, DIM)


# speedup vs baseline: 1.2255x; 1.2255x over previous
import functools

import jax
import jax.numpy as jnp
from jax import lax
from jax.experimental import pallas as pl
from jax.experimental.pallas import tpu as pltpu
from jax.experimental.pallas import tpu_sc as plsc

DIM = 32
ROWS = 16384
COLS = 26
NW = 32
IB = ROWS // NW
SKEW = IB + 1

_mesh = plsc.VectorSubcoreMesh(core_axis_name="c", subcore_axis_name="s")


@functools.partial(
    pl.kernel,
    mesh=_mesh,
    out_type=jax.ShapeDtypeStruct((COLS, DIM, ROWS), jnp.float32),
    scratch_types=[
        pltpu.VMEM((IB,), jnp.int32),
        pltpu.VMEM((IB,), jnp.int32),
        pltpu.VMEM((IB, DIM), jnp.float32),
        pltpu.VMEM((IB, DIM), jnp.float32),
        pltpu.VMEM((DIM, SKEW), jnp.float32),
        pltpu.VMEM((DIM, SKEW), jnp.float32),
        pltpu.SemaphoreType.DMA((2,)),
        pltpu.SemaphoreType.DMA((2,)),
    ],
    compiler_params=pltpu.CompilerParams(
        use_tc_tiling_on_sc=False, needs_layout_passes=False
    ),
)
def _emb_lookup(xT_hbm, table_hbm, out_hbm, idx0, idx1, rows0, rows1,
                tb0, tb1, gsem, wsem):
    wid = lax.axis_index("s") * 2 + lax.axis_index("c")
    i0 = wid * IB
    idx_b = (idx0, idx1)
    rows_b = (rows0, rows1)
    tb_b = (tb0, tb1)
    i16 = lax.iota(jnp.int32, 16)

    def fire_gather(q, b):
        idx = idx_b[b]
        pltpu.sync_copy(xT_hbm.at[q, pl.ds(i0, IB)], idx)

        def scale(k, carry):
            idx[pl.ds(k * 16, 16)] = idx[pl.ds(k * 16, 16)] * 4
            return carry

        lax.fori_loop(0, IB // 16, scale, 0)
        return pltpu.async_copy(table_hbm.at[idx], rows_b[b], gsem.at[b])

    def wait_gather(b):
        pltpu.make_async_copy(
            table_hbm.at[idx_b[b]], rows_b[b], gsem.at[b]
        ).wait()

    def fire_write(q, b):
        return pltpu.async_copy(
            tb_b[b].at[:, pl.ds(0, IB)],
            out_hbm.at[q, :, pl.ds(i0, IB)],
            wsem.at[b],
        )

    def wait_write(q, b):
        pltpu.make_async_copy(
            tb_b[b].at[:, pl.ds(0, IB)],
            out_hbm.at[q, :, pl.ds(i0, IB)],
            wsem.at[b],
        ).wait()

    def transpose(b):
        rows = rows_b[b]
        tb = tb_b[b]

        def pair(j2, carry):
            for u in range(2):
                j = j2 * 2 + u
                jv = jnp.full((16,), 0, jnp.int32) + j
                v0 = rows[j, pl.ds(0, 16)]
                v1 = rows[j, pl.ds(16, 16)]
                plsc.store_scatter(tb, [i16, jv], v0)
                plsc.store_scatter(tb, [i16 + 16, jv], v1)
            return carry

        lax.fori_loop(0, IB // 2, pair, 0)

    fire_gather(0, 0)
    fire_gather(1, 1)

    for u in range(2):
        wait_gather(u)
        transpose(u)
        fire_gather(u + 2, u)
        fire_write(u, u)

    def body(t, carry):
        for u in range(2):
            q = 2 * t + u
            wait_gather(u)
            wait_write(q - 2, u)
            transpose(u)
            fire_gather(q + 2, u)
            fire_write(q, u)
        return carry

    lax.fori_loop(1, 12, body, 0)

    for u in range(2):
        q = 24 + u
        wait_gather(u)
        wait_write(q - 2, u)
        transpose(u)
        fire_write(q, u)
    for u in range(2):
        wait_write(24 + u, u)


def kernel(x, table):
    t_pad = jnp.pad(table, ((0, 0), (0, 128 - DIM))).reshape(4 * 1000000, DIM)
    outT = _emb_lookup(x.T, t_pad)
    return outT.transpose(2, 0, 1)
